# baseline (speedup 1.0000x reference)
import functools

import jax
import jax.numpy as jnp
import numpy as np
from jax import lax
from jax.experimental import pallas as pl
from jax.experimental.pallas import tpu as pltpu
from jax.experimental.pallas import tpu_sc as plsc
from jax._src.random import threefry2x32 as _threefry

_NV = 1000
_ROWS = 512
_NROWS = 65536
_R0 = 65536
_NSC = _NROWS - _R0
_TINY = np.float32(np.finfo(np.float32).tiny)

_NWORK = 32
_GROWS = 16
_ROWS_W = _NSC // _NWORK


def _tc_body(z_ref, out_ref, *, row_base):
    rows = out_ref.shape[0]
    base = (row_base + pl.program_id(0) * rows) * _NV
    row = lax.broadcasted_iota(jnp.int32, (rows, _NV), 0)
    col = lax.broadcasted_iota(jnp.int32, (rows, _NV), 1)
    cnt = (base + row * _NV + col).astype(jnp.uint32)
    bits = _threefry_bits(cnt)
    fb = (bits >> jnp.uint32(9)) | jnp.uint32(0x3F800000)
    f = lax.bitcast_convert_type(fb, jnp.float32) - jnp.float32(1.0)
    u = jnp.maximum(f, _TINY)
    g = -jnp.log(-jnp.log(u))
    score = z_ref[...] + g
    mx = jnp.max(score, axis=1, keepdims=True)
    idx = jnp.min(jnp.where(score == mx, col, _NV), axis=1, keepdims=True)
    out_ref[...] = idx


_ROT_A = (13, 15, 26, 6)
_ROT_B = (17, 29, 16, 24)
_KS0 = np.uint32(0)
_KS1 = np.uint32(42)
_KS2 = np.uint32(42 ^ 0x1BD11BDA)
_SQRT2 = np.float32(1.4142135623730951)
_LN2_HI = np.float32(0.69314575195)
_LN2_LO = np.float32(1.4286067653e-06)


def _rotl(v, r):
    return lax.shift_left(v, jnp.uint32(r)) | lax.shift_right_logical(
        v, jnp.uint32(32 - r))


def _round(x0, x1, r):
    x0 = x0 + x1
    x1 = _rotl(x1, r) ^ x0
    return x0, x1


def _threefry_bits(cnt):
    y = cnt + _KS1
    x0 = y
    x1 = _rotl(y, _ROT_A[0]) ^ y
    for r in _ROT_A[1:]:
        x0, x1 = _round(x0, x1, r)
    x0 = x0 + _KS1
    x1 = x1 + (_KS2 + np.uint32(1))
    for r in _ROT_B:
        x0, x1 = _round(x0, x1, r)
    x0 = x0 + _KS2
    x1 = x1 + (_KS0 + np.uint32(2))
    for r in _ROT_A:
        x0, x1 = _round(x0, x1, r)
    x0 = x0 + _KS0
    x1 = x1 + (_KS1 + np.uint32(3))
    for r in _ROT_B:
        x0, x1 = _round(x0, x1, r)
    x0 = x0 + _KS1
    x1 = x1 + (_KS2 + np.uint32(4))
    for r in _ROT_A:
        x0, x1 = _round(x0, x1, r)
    x0 = x0 + _KS2
    x1 = x1 + (_KS0 + np.uint32(5))
    return x0 ^ x1


def _neg_ln(u):
    ib = lax.bitcast_convert_type(u, jnp.int32)
    e = (ib >> 23) - 127
    m = lax.bitcast_convert_type(
        (ib & jnp.int32(0x7FFFFF)) | jnp.int32(0x3F800000), jnp.float32)
    big = m > _SQRT2
    m = jnp.where(big, m * jnp.float32(0.5), m)
    e = jnp.where(big, e + 1, e)
    ef = e.astype(jnp.float32)
    s = (m - jnp.float32(1.0)) / (m + jnp.float32(1.0))
    s2 = s * s
    p = jnp.float32(1.0 / 9.0)
    for coef in (1.0 / 7.0, 1.0 / 5.0, 1.0 / 3.0, 1.0):
        p = p * s2 + jnp.float32(coef)
    lnm = jnp.float32(2.0) * s * p
    return -(ef * _LN2_HI + (lnm + ef * _LN2_LO))


_UNROLL = 2


def _sc_body(zf, out, buf, outbuf, dsem):
    nc = 2
    wid = lax.axis_index("s") * nc + lax.axis_index("c")
    row0 = _R0 + wid * _ROWS_W
    lanes = lax.iota(jnp.int32, 16)
    lanebase = lanes * _NV
    ngroups = _ROWS_W // _GROWS

    def group_body(gi, _):
        gbase = (row0 + gi * _GROWS) * _NV
        cp = pltpu.make_async_copy(
            zf.at[pl.ds(gbase, _GROWS * _NV)], buf, dsem)
        cp.start()
        cp.wait()
        cnt0 = (gbase + lanebase).astype(jnp.uint32)

        def step(kk, carry):
            a_best, t_best, i_best = carry
            for uu in range(_UNROLL):
                k = kk * _UNROLL + uu
                z16 = plsc.load_gather(buf, [lanebase + k])
                bits = _threefry_bits(cnt0 + k.astype(jnp.uint32))
                fb = (bits >> jnp.uint32(9)) | jnp.uint32(0x3F800000)
                f = lax.bitcast_convert_type(fb, jnp.float32) - 1.0
                u = jnp.maximum(f, _TINY)
                t = _neg_ln(u)
                a = jnp.exp(z16)
                better = a * t_best > a_best * t
                a_best = jnp.where(better, a, a_best)
                t_best = jnp.where(better, t, t_best)
                i_best = jnp.where(better, k, i_best)
            return a_best, t_best, i_best

        a_best = jnp.zeros((16,), jnp.float32)
        t_best = jnp.ones((16,), jnp.float32)
        i_best = jnp.zeros((16,), jnp.int32)
        _, _, i_best = lax.fori_loop(
            0, _NV // _UNROLL, step, (a_best, t_best, i_best))
        outbuf[pl.ds(gi * _GROWS, 16)] = i_best
        return 0

    lax.fori_loop(0, ngroups, group_body, 0)
    pltpu.sync_copy(outbuf, out.at[pl.ds(wid * _ROWS_W, _ROWS_W)])


def _sc_call(zf):
    mesh = plsc.VectorSubcoreMesh(core_axis_name="c", subcore_axis_name="s")
    return pl.kernel(
        _sc_body,
        out_type=jax.ShapeDtypeStruct((_NSC,), jnp.int32),
        mesh=mesh,
        scratch_types=[
            pltpu.VMEM((_GROWS * _NV,), jnp.float32),
            pltpu.VMEM((_ROWS_W,), jnp.int32),
            pltpu.SemaphoreType.DMA,
        ],
        compiler_params=pltpu.CompilerParams(needs_layout_passes=False),
        cost_estimate=pl.CostEstimate(
            flops=int(130 * _NSC * _NV),
            bytes_accessed=int(_NSC * _NV * 4),
            transcendentals=int(_NSC * _NV),
        ),
    )(zf)


def kernel(z):
    b = z.shape[0]
    zf = z.reshape(_NROWS * _NV)
    parts = []
    if _NSC:
        parts.append(_sc_call(zf))
    zr = z.reshape(_NROWS, _NV)
    out_tc = pl.pallas_call(
        functools.partial(_tc_body, row_base=0),
        grid=(_R0 // _ROWS,),
        in_specs=[pl.BlockSpec((_ROWS, _NV), lambda i: (i, 0))],
        out_specs=pl.BlockSpec((_ROWS, 1), lambda i: (i, 0)),
        out_shape=jax.ShapeDtypeStruct((_R0, 1), jnp.int32),
        compiler_params=pltpu.CompilerParams(
            dimension_semantics=("parallel",),
        ),
    )(zr)
    out = jnp.concatenate([out_tc.reshape(_R0)] + parts)
    return out.reshape(b, _NROWS // b)


# speedup vs baseline: 1.0038x; 1.0038x over previous
import functools

import jax
import jax.numpy as jnp
import numpy as np
from jax import lax
from jax.experimental import pallas as pl
from jax.experimental.pallas import tpu as pltpu
from jax.experimental.pallas import tpu_sc as plsc
from jax._src.random import threefry2x32 as _threefry

_NV = 1000
_ROWS = 1024
_NROWS = 65536
_R0 = 65536
_NSC = _NROWS - _R0
_TINY = np.float32(np.finfo(np.float32).tiny)

_NWORK = 32
_GROWS = 16
_ROWS_W = _NSC // _NWORK


def _tc_body(z_ref, out_ref, *, row_base):
    rows = out_ref.shape[0]
    base = (row_base + pl.program_id(0) * rows) * _NV
    row = lax.broadcasted_iota(jnp.int32, (rows, _NV), 0)
    col = lax.broadcasted_iota(jnp.int32, (rows, _NV), 1)
    cnt = (base + row * _NV + col).astype(jnp.uint32)
    bits = _threefry_bits(cnt)
    fb = (bits >> jnp.uint32(9)) | jnp.uint32(0x3F800000)
    f = lax.bitcast_convert_type(fb, jnp.float32) - jnp.float32(1.0)
    u = jnp.maximum(f, _TINY)
    g = -jnp.log(-jnp.log(u))
    score = z_ref[...] + g
    mx = jnp.max(score, axis=1, keepdims=True)
    idx = jnp.min(jnp.where(score == mx, col, _NV), axis=1, keepdims=True)
    out_ref[...] = idx


_ROT_A = (13, 15, 26, 6)
_ROT_B = (17, 29, 16, 24)
_KS0 = np.uint32(0)
_KS1 = np.uint32(42)
_KS2 = np.uint32(42 ^ 0x1BD11BDA)
_SQRT2 = np.float32(1.4142135623730951)
_LN2_HI = np.float32(0.69314575195)
_LN2_LO = np.float32(1.4286067653e-06)


def _rotl(v, r):
    return lax.shift_left(v, jnp.uint32(r)) | lax.shift_right_logical(
        v, jnp.uint32(32 - r))


def _round(x0, x1, r):
    x0 = x0 + x1
    x1 = _rotl(x1, r) ^ x0
    return x0, x1


def _threefry_bits(cnt):
    y = cnt + _KS1
    x0 = y
    x1 = _rotl(y, _ROT_A[0]) ^ y
    for r in _ROT_A[1:]:
        x0, x1 = _round(x0, x1, r)
    x0 = x0 + _KS1
    x1 = x1 + (_KS2 + np.uint32(1))
    for r in _ROT_B:
        x0, x1 = _round(x0, x1, r)
    x0 = x0 + _KS2
    x1 = x1 + (_KS0 + np.uint32(2))
    for r in _ROT_A:
        x0, x1 = _round(x0, x1, r)
    x0 = x0 + _KS0
    x1 = x1 + (_KS1 + np.uint32(3))
    for r in _ROT_B:
        x0, x1 = _round(x0, x1, r)
    x0 = x0 + _KS1
    x1 = x1 + (_KS2 + np.uint32(4))
    for r in _ROT_A:
        x0, x1 = _round(x0, x1, r)
    x0 = x0 + _KS2
    x1 = x1 + (_KS0 + np.uint32(5))
    return x0 ^ x1


def _neg_ln(u):
    ib = lax.bitcast_convert_type(u, jnp.int32)
    e = (ib >> 23) - 127
    m = lax.bitcast_convert_type(
        (ib & jnp.int32(0x7FFFFF)) | jnp.int32(0x3F800000), jnp.float32)
    big = m > _SQRT2
    m = jnp.where(big, m * jnp.float32(0.5), m)
    e = jnp.where(big, e + 1, e)
    ef = e.astype(jnp.float32)
    s = (m - jnp.float32(1.0)) / (m + jnp.float32(1.0))
    s2 = s * s
    p = jnp.float32(1.0 / 9.0)
    for coef in (1.0 / 7.0, 1.0 / 5.0, 1.0 / 3.0, 1.0):
        p = p * s2 + jnp.float32(coef)
    lnm = jnp.float32(2.0) * s * p
    return -(ef * _LN2_HI + (lnm + ef * _LN2_LO))


_UNROLL = 2


def _sc_body(zf, out, buf, outbuf, dsem):
    nc = 2
    wid = lax.axis_index("s") * nc + lax.axis_index("c")
    row0 = _R0 + wid * _ROWS_W
    lanes = lax.iota(jnp.int32, 16)
    lanebase = lanes * _NV
    ngroups = _ROWS_W // _GROWS

    def group_body(gi, _):
        gbase = (row0 + gi * _GROWS) * _NV
        cp = pltpu.make_async_copy(
            zf.at[pl.ds(gbase, _GROWS * _NV)], buf, dsem)
        cp.start()
        cp.wait()
        cnt0 = (gbase + lanebase).astype(jnp.uint32)

        def step(kk, carry):
            a_best, t_best, i_best = carry
            for uu in range(_UNROLL):
                k = kk * _UNROLL + uu
                z16 = plsc.load_gather(buf, [lanebase + k])
                bits = _threefry_bits(cnt0 + k.astype(jnp.uint32))
                fb = (bits >> jnp.uint32(9)) | jnp.uint32(0x3F800000)
                f = lax.bitcast_convert_type(fb, jnp.float32) - 1.0
                u = jnp.maximum(f, _TINY)
                t = _neg_ln(u)
                a = jnp.exp(z16)
                better = a * t_best > a_best * t
                a_best = jnp.where(better, a, a_best)
                t_best = jnp.where(better, t, t_best)
                i_best = jnp.where(better, k, i_best)
            return a_best, t_best, i_best

        a_best = jnp.zeros((16,), jnp.float32)
        t_best = jnp.ones((16,), jnp.float32)
        i_best = jnp.zeros((16,), jnp.int32)
        _, _, i_best = lax.fori_loop(
            0, _NV // _UNROLL, step, (a_best, t_best, i_best))
        outbuf[pl.ds(gi * _GROWS, 16)] = i_best
        return 0

    lax.fori_loop(0, ngroups, group_body, 0)
    pltpu.sync_copy(outbuf, out.at[pl.ds(wid * _ROWS_W, _ROWS_W)])


def _sc_call(zf):
    mesh = plsc.VectorSubcoreMesh(core_axis_name="c", subcore_axis_name="s")
    return pl.kernel(
        _sc_body,
        out_type=jax.ShapeDtypeStruct((_NSC,), jnp.int32),
        mesh=mesh,
        scratch_types=[
            pltpu.VMEM((_GROWS * _NV,), jnp.float32),
            pltpu.VMEM((_ROWS_W,), jnp.int32),
            pltpu.SemaphoreType.DMA,
        ],
        compiler_params=pltpu.CompilerParams(needs_layout_passes=False),
        cost_estimate=pl.CostEstimate(
            flops=int(130 * _NSC * _NV),
            bytes_accessed=int(_NSC * _NV * 4),
            transcendentals=int(_NSC * _NV),
        ),
    )(zf)


def kernel(z):
    b = z.shape[0]
    zf = z.reshape(_NROWS * _NV)
    parts = []
    if _NSC:
        parts.append(_sc_call(zf))
    zr = z.reshape(_NROWS, _NV)
    out_tc = pl.pallas_call(
        functools.partial(_tc_body, row_base=0),
        grid=(_R0 // _ROWS,),
        in_specs=[pl.BlockSpec((_ROWS, _NV), lambda i: (i, 0))],
        out_specs=pl.BlockSpec((_ROWS, 1), lambda i: (i, 0)),
        out_shape=jax.ShapeDtypeStruct((_R0, 1), jnp.int32),
        compiler_params=pltpu.CompilerParams(
            dimension_semantics=("parallel",),
        ),
    )(zr)
    out = jnp.concatenate([out_tc.reshape(_R0)] + parts)
    return out.reshape(b, _NROWS // b)


# speedup vs baseline: 1.0039x; 1.0001x over previous
import functools

import jax
import jax.numpy as jnp
import numpy as np
from jax import lax
from jax.experimental import pallas as pl
from jax.experimental.pallas import tpu as pltpu
from jax.experimental.pallas import tpu_sc as plsc

_NV = 1000
_ROWS = 1024
_NROWS = 65536
_R0 = 65536
_NSC = _NROWS - _R0
_TINY = np.float32(np.finfo(np.float32).tiny)

_NWORK = 32
_GROWS = 16
_ROWS_W = _NSC // _NWORK


def _tc_body(z_ref, out_ref, *, row_base):
    rows = out_ref.shape[0]
    base = (row_base + pl.program_id(0) * rows) * _NV
    row = lax.broadcasted_iota(jnp.int32, (rows, _NV), 0)
    col = lax.broadcasted_iota(jnp.int32, (rows, _NV), 1)
    cnt = (base + row * _NV + col).astype(jnp.uint32)
    bits = _threefry_bits(cnt)
    fb = (bits >> jnp.uint32(9)) | jnp.uint32(0x3F800000)
    f = lax.bitcast_convert_type(fb, jnp.float32) - jnp.float32(1.0)
    u = jnp.maximum(f, _TINY)
    g = -jnp.log(-jnp.log(u))
    score = z_ref[...] + g
    mx = jnp.max(score, axis=1, keepdims=True)
    idx = jnp.min(jnp.where(score == mx, col, _NV), axis=1, keepdims=True)
    out_ref[...] = idx


_ROT_A = (13, 15, 26, 6)
_ROT_B = (17, 29, 16, 24)
_KS0 = np.uint32(0)
_KS1 = np.uint32(42)
_KS2 = np.uint32(42 ^ 0x1BD11BDA)
_SQRT2 = np.float32(1.4142135623730951)
_LN2_HI = np.float32(0.69314575195)
_LN2_LO = np.float32(1.4286067653e-06)


def _rotl(v, r):
    return lax.shift_left(v, jnp.uint32(r)) | lax.shift_right_logical(
        v, jnp.uint32(32 - r))


def _round(x0, x1, r):
    x0 = x0 + x1
    x1 = _rotl(x1, r) ^ x0
    return x0, x1


def _threefry_bits(cnt):
    y = cnt + _KS1
    x0 = y
    x1 = _rotl(y, _ROT_A[0]) ^ y
    for r in _ROT_A[1:]:
        x0, x1 = _round(x0, x1, r)
    x0 = x0 + _KS1
    x1 = x1 + (_KS2 + np.uint32(1))
    for r in _ROT_B:
        x0, x1 = _round(x0, x1, r)
    x0 = x0 + _KS2
    x1 = x1 + (_KS0 + np.uint32(2))
    for r in _ROT_A:
        x0, x1 = _round(x0, x1, r)
    x0 = x0 + _KS0
    x1 = x1 + (_KS1 + np.uint32(3))
    for r in _ROT_B:
        x0, x1 = _round(x0, x1, r)
    x0 = x0 + _KS1
    x1 = x1 + (_KS2 + np.uint32(4))
    for r in _ROT_A:
        x0, x1 = _round(x0, x1, r)
    x0 = x0 + _KS2
    x1 = x1 + (_KS0 + np.uint32(5))
    return x0 ^ x1


def _neg_ln(u):
    ib = lax.bitcast_convert_type(u, jnp.int32)
    e = (ib >> 23) - 127
    m = lax.bitcast_convert_type(
        (ib & jnp.int32(0x7FFFFF)) | jnp.int32(0x3F800000), jnp.float32)
    big = m > _SQRT2
    m = jnp.where(big, m * jnp.float32(0.5), m)
    e = jnp.where(big, e + 1, e)
    ef = e.astype(jnp.float32)
    s = (m - jnp.float32(1.0)) / (m + jnp.float32(1.0))
    s2 = s * s
    p = jnp.float32(1.0 / 9.0)
    for coef in (1.0 / 7.0, 1.0 / 5.0, 1.0 / 3.0, 1.0):
        p = p * s2 + jnp.float32(coef)
    lnm = jnp.float32(2.0) * s * p
    return -(ef * _LN2_HI + (lnm + ef * _LN2_LO))


_UNROLL = 2


def _sc_body(zf, out, buf, outbuf, dsem):
    nc = 2
    wid = lax.axis_index("s") * nc + lax.axis_index("c")
    row0 = _R0 + wid * _ROWS_W
    lanes = lax.iota(jnp.int32, 16)
    lanebase = lanes * _NV
    ngroups = _ROWS_W // _GROWS

    def group_body(gi, _):
        gbase = (row0 + gi * _GROWS) * _NV
        cp = pltpu.make_async_copy(
            zf.at[pl.ds(gbase, _GROWS * _NV)], buf, dsem)
        cp.start()
        cp.wait()
        cnt0 = (gbase + lanebase).astype(jnp.uint32)

        def step(kk, carry):
            a_best, t_best, i_best = carry
            for uu in range(_UNROLL):
                k = kk * _UNROLL + uu
                z16 = plsc.load_gather(buf, [lanebase + k])
                bits = _threefry_bits(cnt0 + k.astype(jnp.uint32))
                fb = (bits >> jnp.uint32(9)) | jnp.uint32(0x3F800000)
                f = lax.bitcast_convert_type(fb, jnp.float32) - 1.0
                u = jnp.maximum(f, _TINY)
                t = _neg_ln(u)
                a = jnp.exp(z16)
                better = a * t_best > a_best * t
                a_best = jnp.where(better, a, a_best)
                t_best = jnp.where(better, t, t_best)
                i_best = jnp.where(better, k, i_best)
            return a_best, t_best, i_best

        a_best = jnp.zeros((16,), jnp.float32)
        t_best = jnp.ones((16,), jnp.float32)
        i_best = jnp.zeros((16,), jnp.int32)
        _, _, i_best = lax.fori_loop(
            0, _NV // _UNROLL, step, (a_best, t_best, i_best))
        outbuf[pl.ds(gi * _GROWS, 16)] = i_best
        return 0

    lax.fori_loop(0, ngroups, group_body, 0)
    pltpu.sync_copy(outbuf, out.at[pl.ds(wid * _ROWS_W, _ROWS_W)])


def _sc_call(zf):
    mesh = plsc.VectorSubcoreMesh(core_axis_name="c", subcore_axis_name="s")
    return pl.kernel(
        _sc_body,
        out_type=jax.ShapeDtypeStruct((_NSC,), jnp.int32),
        mesh=mesh,
        scratch_types=[
            pltpu.VMEM((_GROWS * _NV,), jnp.float32),
            pltpu.VMEM((_ROWS_W,), jnp.int32),
            pltpu.SemaphoreType.DMA,
        ],
        compiler_params=pltpu.CompilerParams(needs_layout_passes=False),
        cost_estimate=pl.CostEstimate(
            flops=int(130 * _NSC * _NV),
            bytes_accessed=int(_NSC * _NV * 4),
            transcendentals=int(_NSC * _NV),
        ),
    )(zf)


def kernel(z):
    b = z.shape[0]
    zf = z.reshape(_NROWS * _NV)
    parts = []
    if _NSC:
        parts.append(_sc_call(zf))
    zr = z.reshape(_NROWS, _NV)
    out_tc = pl.pallas_call(
        functools.partial(_tc_body, row_base=0),
        grid=(_R0 // _ROWS,),
        in_specs=[pl.BlockSpec((_ROWS, _NV), lambda i: (i, 0))],
        out_specs=pl.BlockSpec((_ROWS, 1), lambda i: (i, 0)),
        out_shape=jax.ShapeDtypeStruct((_R0, 1), jnp.int32),
        compiler_params=pltpu.CompilerParams(
            dimension_semantics=("parallel",),
        ),
    )(zr)
    out = jnp.concatenate([out_tc.reshape(_R0)] + parts)
    return out.reshape(b, _NROWS // b)
